# baseline (device time: 23051 ns/iter reference)
import jax
import jax.numpy as jnp
from jax import lax
from jax.experimental import pallas as pl
from jax.experimental.pallas import tpu as pltpu

N_DEV = 8
BLK = 64


def kernel(x, Wq, K_ext, V_ext, Wo):
    B, Sq, Dm = x.shape
    _, Skv, Hq, Dh = K_ext.shape
    H_loc = Wq.shape[1] // Dh
    HD = H_loc * Dh
    R = B * Sq
    R_BLK = R // N_DEV
    BPB = Sq // R_BLK

    xf = x.reshape(R, Dm)
    kf = K_ext.reshape(B, Skv, Hq * Dh)
    vf = V_ext.reshape(B, Skv, Hq * Dh)

    def body(x_ref, wq_ref, k_hbm, v_hbm, wo_ref, out_ref,
             st_ref, rs_ref, ag_ref, kv_ref, kv_sems,
             rs_send_sems, rs_recv_sems, ag_send_sems, ag_recv_sems):
        my_pos = lax.axis_index("i")
        c0 = my_pos * HD

        kv_cps = []
        for t, src in ((0, k_hbm), (1, v_hbm)):
            for b in range(B):
                cp = pltpu.make_async_copy(
                    src.at[b, :, pl.ds(c0, HD)],
                    kv_ref.at[t, b],
                    kv_sems.at[t * B + b],
                )
                cp.start()
                kv_cps.append(cp)

        barrier_sem = pltpu.get_barrier_semaphore()
        for d in range(1, N_DEV):
            pl.semaphore_signal(
                barrier_sem, inc=1,
                device_id=((my_pos + d) % N_DEV,),
                device_id_type=pl.DeviceIdType.MESH,
            )

        wq_bf = wq_ref[...].astype(jnp.bfloat16)
        wo_bf = wo_ref[...].astype(jnp.bfloat16)
        rows = lax.broadcasted_iota(jnp.int32, (Sq, Skv), 0) // BLK
        cols = lax.broadcasted_iota(jnp.int32, (Sq, Skv), 1) // BLK
        mask = cols <= rows

        for b in range(B):
            qb = jnp.dot(x_ref[pl.ds(b * Sq, Sq), :].astype(jnp.bfloat16),
                         wq_bf,
                         preferred_element_type=jnp.float32)
            qb = qb.astype(jnp.bfloat16)
            kv_cps[b].wait()
            kv_cps[B + b].wait()
            kb = kv_ref[0, b].astype(jnp.bfloat16)
            vb = kv_ref[1, b].astype(jnp.bfloat16)
            ctx_heads = []
            for h in range(H_loc):
                s = lax.dot_general(
                    qb[:, h * Dh:(h + 1) * Dh], kb[:, h * Dh:(h + 1) * Dh],
                    (((1,), (1,)), ((), ())),
                    preferred_element_type=jnp.float32,
                ) * 0.125
                s = jnp.where(mask, s, -1e9)
                mx = jnp.max(s, axis=-1, keepdims=True)
                w = jnp.exp(s - mx)
                w = w / jnp.sum(w, axis=-1, keepdims=True)
                ctx_heads.append(
                    jnp.dot(w.astype(jnp.bfloat16), vb[:, h * Dh:(h + 1) * Dh],
                            preferred_element_type=jnp.float32)
                    .astype(jnp.bfloat16))
            ctx_b = jnp.concatenate(ctx_heads, axis=1)
            partial_b = jnp.dot(ctx_b, wo_bf,
                                preferred_element_type=jnp.float32)
            out_ref[pl.ds(b * Sq, Sq), :] = partial_b
            st_ref[pl.ds(b * Sq, Sq), :] = partial_b.astype(jnp.bfloat16)

            if b == 0:
                pl.semaphore_wait(barrier_sem, N_DEV - 1)

            for j in range(b * BPB, (b + 1) * BPB):
                @pl.when(j != my_pos)
                def _(j=j):
                    rdma = pltpu.make_async_remote_copy(
                        src_ref=st_ref.at[pl.ds(j * R_BLK, R_BLK)],
                        dst_ref=rs_ref.at[my_pos],
                        send_sem=rs_send_sems.at[j],
                        recv_sem=rs_recv_sems.at[my_pos],
                        device_id=(j,),
                        device_id_type=pl.DeviceIdType.MESH,
                    )
                    rdma.start()

        acc = out_ref[pl.ds(my_pos * R_BLK, R_BLK), :]
        for d in range(1, N_DEV):
            i = (my_pos + d) % N_DEV
            recv = pltpu.make_async_remote_copy(
                src_ref=rs_ref.at[i],
                dst_ref=rs_ref.at[i],
                send_sem=rs_send_sems.at[i],
                recv_sem=rs_recv_sems.at[i],
                device_id=(i,),
                device_id_type=pl.DeviceIdType.MESH,
            )
            recv.wait_recv()
            acc = acc + rs_ref[i].astype(jnp.float32)

        out_ref[pl.ds(my_pos * R_BLK, R_BLK), :] = acc
        ag_ref[pl.ds(my_pos * R_BLK, R_BLK), :] = acc.astype(jnp.bfloat16)
        for d in range(1, N_DEV):
            j = (my_pos + d) % N_DEV
            rdma = pltpu.make_async_remote_copy(
                src_ref=ag_ref.at[pl.ds(my_pos * R_BLK, R_BLK)],
                dst_ref=ag_ref.at[pl.ds(my_pos * R_BLK, R_BLK)],
                send_sem=ag_send_sems.at[j],
                recv_sem=ag_recv_sems.at[my_pos],
                device_id=(j,),
                device_id_type=pl.DeviceIdType.MESH,
            )
            rdma.start()

        for d in range(1, N_DEV):
            i = (my_pos + d) % N_DEV
            recv = pltpu.make_async_remote_copy(
                src_ref=ag_ref.at[pl.ds(i * R_BLK, R_BLK)],
                dst_ref=ag_ref.at[pl.ds(i * R_BLK, R_BLK)],
                send_sem=ag_send_sems.at[i],
                recv_sem=ag_recv_sems.at[i],
                device_id=(i,),
                device_id_type=pl.DeviceIdType.MESH,
            )
            recv.wait_recv()
            out_ref[pl.ds(i * R_BLK, R_BLK), :] = (
                ag_ref[pl.ds(i * R_BLK, R_BLK), :].astype(jnp.float32))

        for d in range(1, N_DEV):
            j = (my_pos + d) % N_DEV
            send = pltpu.make_async_remote_copy(
                src_ref=st_ref.at[pl.ds(j * R_BLK, R_BLK)],
                dst_ref=rs_ref.at[my_pos],
                send_sem=rs_send_sems.at[j],
                recv_sem=rs_recv_sems.at[my_pos],
                device_id=(j,),
                device_id_type=pl.DeviceIdType.MESH,
            )
            send.wait_send()
            send2 = pltpu.make_async_remote_copy(
                src_ref=ag_ref.at[pl.ds(my_pos * R_BLK, R_BLK)],
                dst_ref=ag_ref.at[pl.ds(my_pos * R_BLK, R_BLK)],
                send_sem=ag_send_sems.at[j],
                recv_sem=ag_recv_sems.at[my_pos],
                device_id=(j,),
                device_id_type=pl.DeviceIdType.MESH,
            )
            send2.wait_send()

    out = pl.pallas_call(
        body,
        out_shape=jax.ShapeDtypeStruct((R, Dm), jnp.float32),
        in_specs=[
            pl.BlockSpec(memory_space=pltpu.VMEM),
            pl.BlockSpec(memory_space=pltpu.VMEM),
            pl.BlockSpec(memory_space=pltpu.MemorySpace.HBM),
            pl.BlockSpec(memory_space=pltpu.MemorySpace.HBM),
            pl.BlockSpec(memory_space=pltpu.VMEM),
        ],
        out_specs=pl.BlockSpec(memory_space=pltpu.VMEM),
        scratch_shapes=[
            pltpu.VMEM((R, Dm), jnp.bfloat16),
            pltpu.VMEM((N_DEV, R_BLK, Dm), jnp.bfloat16),
            pltpu.VMEM((R, Dm), jnp.bfloat16),
            pltpu.VMEM((2, B, Skv, HD), jnp.float32),
            pltpu.SemaphoreType.DMA((2 * B,)),
            pltpu.SemaphoreType.DMA((N_DEV,)),
            pltpu.SemaphoreType.DMA((N_DEV,)),
            pltpu.SemaphoreType.DMA((N_DEV,)),
            pltpu.SemaphoreType.DMA((N_DEV,)),
        ],
        compiler_params=pltpu.CompilerParams(collective_id=0),
    )(xf, Wq, kf, vf, Wo)

    return out.reshape(B, Sq, Dm)


# device time: 20730 ns/iter; 1.1120x vs baseline; 1.1120x over previous
import jax
import jax.numpy as jnp
from jax import lax
from jax.experimental import pallas as pl
from jax.experimental.pallas import tpu as pltpu

N_DEV = 8
BLK = 64


def kernel(x, Wq, K_ext, V_ext, Wo):
    B, Sq, Dm = x.shape
    _, Skv, _, Dh = K_ext.shape
    H_loc = Wq.shape[1] // Dh
    BH = B * H_loc
    R = B * Sq
    R_BLK = R // N_DEV
    BPB = Sq // R_BLK

    my = lax.axis_index("i")

    xb = x.reshape(R, Dm)
    k_loc = lax.dynamic_slice_in_dim(K_ext, my * H_loc, H_loc, axis=2)
    v_loc = lax.dynamic_slice_in_dim(V_ext, my * H_loc, H_loc, axis=2)
    k_loc = k_loc.reshape(B, Skv, H_loc * Dh).astype(jnp.bfloat16)
    v_loc = v_loc.reshape(B, Skv, H_loc * Dh).astype(jnp.bfloat16)

    def body(x_ref, wq_ref, k_ref, v_ref, wo_ref, out_ref,
             st_ref, rs_ref, ag_ref,
             rs_send_sems, rs_recv_sems, ag_send_sems, ag_recv_sems):
        my_pos = lax.axis_index("i")

        barrier_sem = pltpu.get_barrier_semaphore()
        for d in range(1, N_DEV):
            pl.semaphore_signal(
                barrier_sem, inc=1,
                device_id=((my_pos + d) % N_DEV,),
                device_id_type=pl.DeviceIdType.MESH,
            )

        rows = lax.broadcasted_iota(jnp.int32, (Sq, Skv), 0) // BLK
        cols = lax.broadcasted_iota(jnp.int32, (Sq, Skv), 1) // BLK
        mask = cols <= rows
        wq_bf = wq_ref[...].astype(jnp.bfloat16)
        wo_bf = wo_ref[...].astype(jnp.bfloat16)

        for b in range(B):
            qb = jnp.dot(x_ref[pl.ds(b * Sq, Sq), :].astype(jnp.bfloat16),
                         wq_bf,
                         preferred_element_type=jnp.float32)
            qb = qb.astype(jnp.bfloat16)
            kb = k_ref[b]
            vb = v_ref[b]
            ctx_heads = []
            for h in range(H_loc):
                s = lax.dot_general(
                    qb[:, h * Dh:(h + 1) * Dh], kb[:, h * Dh:(h + 1) * Dh],
                    (((1,), (1,)), ((), ())),
                    preferred_element_type=jnp.float32,
                ) * 0.125
                s = jnp.where(mask, s, -1e9)
                mx = jnp.max(s, axis=-1, keepdims=True)
                w = jnp.exp(s - mx)
                w = w / jnp.sum(w, axis=-1, keepdims=True)
                ctx_heads.append(
                    jnp.dot(w.astype(jnp.bfloat16), vb[:, h * Dh:(h + 1) * Dh],
                            preferred_element_type=jnp.float32)
                    .astype(jnp.bfloat16))
            ctx_b = jnp.concatenate(ctx_heads, axis=1)
            partial_b = jnp.dot(ctx_b, wo_bf,
                                preferred_element_type=jnp.float32)
            out_ref[pl.ds(b * Sq, Sq), :] = partial_b
            st_ref[pl.ds(b * Sq, Sq), :] = partial_b.astype(jnp.bfloat16)

            if b == 0:
                pl.semaphore_wait(barrier_sem, N_DEV - 1)

            for j in range(b * BPB, (b + 1) * BPB):
                @pl.when(j != my_pos)
                def _(j=j):
                    rdma = pltpu.make_async_remote_copy(
                        src_ref=st_ref.at[pl.ds(j * R_BLK, R_BLK)],
                        dst_ref=rs_ref.at[my_pos],
                        send_sem=rs_send_sems.at[j],
                        recv_sem=rs_recv_sems.at[my_pos],
                        device_id=(j,),
                        device_id_type=pl.DeviceIdType.MESH,
                    )
                    rdma.start()

        acc = out_ref[pl.ds(my_pos * R_BLK, R_BLK), :]
        for d in range(1, N_DEV):
            i = (my_pos + d) % N_DEV
            recv = pltpu.make_async_remote_copy(
                src_ref=rs_ref.at[i],
                dst_ref=rs_ref.at[i],
                send_sem=rs_send_sems.at[i],
                recv_sem=rs_recv_sems.at[i],
                device_id=(i,),
                device_id_type=pl.DeviceIdType.MESH,
            )
            recv.wait_recv()
            acc = acc + rs_ref[i].astype(jnp.float32)

        out_ref[pl.ds(my_pos * R_BLK, R_BLK), :] = acc
        ag_ref[pl.ds(my_pos * R_BLK, R_BLK), :] = acc.astype(jnp.bfloat16)
        for d in range(1, N_DEV):
            j = (my_pos + d) % N_DEV
            rdma = pltpu.make_async_remote_copy(
                src_ref=ag_ref.at[pl.ds(my_pos * R_BLK, R_BLK)],
                dst_ref=ag_ref.at[pl.ds(my_pos * R_BLK, R_BLK)],
                send_sem=ag_send_sems.at[j],
                recv_sem=ag_recv_sems.at[my_pos],
                device_id=(j,),
                device_id_type=pl.DeviceIdType.MESH,
            )
            rdma.start()

        for d in range(1, N_DEV):
            i = (my_pos + d) % N_DEV
            recv = pltpu.make_async_remote_copy(
                src_ref=ag_ref.at[pl.ds(i * R_BLK, R_BLK)],
                dst_ref=ag_ref.at[pl.ds(i * R_BLK, R_BLK)],
                send_sem=ag_send_sems.at[i],
                recv_sem=ag_recv_sems.at[i],
                device_id=(i,),
                device_id_type=pl.DeviceIdType.MESH,
            )
            recv.wait_recv()
            out_ref[pl.ds(i * R_BLK, R_BLK), :] = (
                ag_ref[pl.ds(i * R_BLK, R_BLK), :].astype(jnp.float32))

        for d in range(1, N_DEV):
            j = (my_pos + d) % N_DEV
            send = pltpu.make_async_remote_copy(
                src_ref=st_ref.at[pl.ds(j * R_BLK, R_BLK)],
                dst_ref=rs_ref.at[my_pos],
                send_sem=rs_send_sems.at[j],
                recv_sem=rs_recv_sems.at[my_pos],
                device_id=(j,),
                device_id_type=pl.DeviceIdType.MESH,
            )
            send.wait_send()
            send2 = pltpu.make_async_remote_copy(
                src_ref=ag_ref.at[pl.ds(my_pos * R_BLK, R_BLK)],
                dst_ref=ag_ref.at[pl.ds(my_pos * R_BLK, R_BLK)],
                send_sem=ag_send_sems.at[j],
                recv_sem=ag_recv_sems.at[my_pos],
                device_id=(j,),
                device_id_type=pl.DeviceIdType.MESH,
            )
            send2.wait_send()

    out = pl.pallas_call(
        body,
        out_shape=jax.ShapeDtypeStruct((R, Dm), jnp.float32),
        in_specs=[pl.BlockSpec(memory_space=pltpu.VMEM)] * 5,
        out_specs=pl.BlockSpec(memory_space=pltpu.VMEM),
        scratch_shapes=[
            pltpu.VMEM((R, Dm), jnp.bfloat16),
            pltpu.VMEM((N_DEV, R_BLK, Dm), jnp.bfloat16),
            pltpu.VMEM((R, Dm), jnp.bfloat16),
            pltpu.SemaphoreType.DMA((N_DEV,)),
            pltpu.SemaphoreType.DMA((N_DEV,)),
            pltpu.SemaphoreType.DMA((N_DEV,)),
            pltpu.SemaphoreType.DMA((N_DEV,)),
        ],
        compiler_params=pltpu.CompilerParams(collective_id=0),
    )(xb, Wq, k_loc, v_loc, Wo)

    return out.reshape(B, Sq, Dm)
